# baseline (speedup 1.0000x reference)
import numpy as np

import jax
import jax.numpy as jnp
from jax.experimental import pallas as pl
from jax.experimental.pallas import tpu as pltpu

_LAYER_DIMS = [(784, 512), (512, 256), (256, 128), (128, 64), (64, 32), (32, 10)]
_CFG = [(4, 2), (2, 2), (2, 2), (2, 2), (2, 2), (2, 4)]

_BB = 2048


def _dotg(a, b, adim, bdim):
    return jax.lax.dot_general(
        a, b, (((adim,), (bdim,)), ((), ())), preferred_element_type=jnp.float32,
        precision=jax.lax.Precision.HIGHEST)


def _layer(cur, li, w, bcol, rw, rbcol):
    din, dout = _LAYER_DIMS[li]
    ig, og = _CFG[li]
    wi = din // ig
    wo = [dout // og] * og
    wo[-1] = dout - (og - 1) * (dout // og)
    woff = np.cumsum([0] + wo)
    first = li == 0
    cdim = 1 if first else 0

    rmask = jax.lax.broadcasted_iota(jnp.int32, w.shape, 0) == 0
    if first:
        rmask |= jax.lax.broadcasted_iota(jnp.int32, w.shape, 1) == 0
    w = jnp.where(rmask, 0.0, w)
    bcol = jnp.where(
        jax.lax.broadcasted_iota(jnp.int32, bcol.shape, 0) == 0, 0.0, bcol)

    scores = _dotg(rw, cur, 1, cdim) + rbcol
    m = jnp.max(scores, axis=0, keepdims=True)
    e = jnp.exp(scores - m)
    pw = e / jnp.sum(e, axis=0, keepdims=True)

    def cslice(i):
        return cur[:, i * wi:(i + 1) * wi] if first else cur[i * wi:(i + 1) * wi, :]

    parts = [_dotg(w[:, i * wi:(i + 1) * wi], cslice(i), 1, cdim)
             for i in range(ig)]

    psum = None
    for i in range(ig):
        s = pw[i * og:(i + 1) * og, :]
        psum = s if psum is None else psum + s

    outs = []
    for j in range(og):
        acc = _dotg(bcol[woff[j]:woff[j + 1], :], psum[j:j + 1, :], 1, 0)
        for i in range(ig):
            acc = acc + parts[i][woff[j]:woff[j + 1], :] * pw[i * og + j:i * og + j + 1, :]
        outs.append(acc)
    out = jnp.concatenate(outs, axis=0)
    if li < 5:
        out = 0.5 * out * (1.0 + jax.lax.erf(out * 0.7071067811865476))
    return out


def _body(x_ref, *refs):
    w_refs = refs[0:6]
    b_refs = refs[6:12]
    rw_refs = refs[12:18]
    rb_refs = refs[18:24]
    o_ref = refs[24]

    cur = x_ref[...]
    for li in range(6):
        cur = _layer(cur, li, w_refs[li][...], b_refs[li][...],
                     rw_refs[li][...], rb_refs[li][...])
    o_ref[...] = jnp.swapaxes(cur, 0, 1)


def kernel(x, fc_w, fc_b, rt_w, rt_b):
    batch = x.shape[0]
    bb = _BB if batch % _BB == 0 else batch

    b_list = [jnp.reshape(v, (-1, 1)) for v in fc_b]
    rb_list = [jnp.reshape(v, (-1, 1)) for v in rt_b]

    full = lambda arr: pl.BlockSpec(arr.shape, lambda i: (0, 0))
    in_specs = [pl.BlockSpec((bb, x.shape[1]), lambda i: (i, 0))]
    operands = [x]
    for group in (list(fc_w), b_list, list(rt_w), rb_list):
        for arr in group:
            in_specs.append(full(arr))
            operands.append(arr)

    return pl.pallas_call(
        _body,
        grid=(batch // bb,),
        in_specs=in_specs,
        out_specs=pl.BlockSpec((bb, 10), lambda i: (i, 0)),
        out_shape=jax.ShapeDtypeStruct((batch, 10), jnp.float32),
        compiler_params=pltpu.CompilerParams(
            dimension_semantics=("parallel",)),
    )(*operands)


# speedup vs baseline: 2.8667x; 2.8667x over previous
import numpy as np

import jax
import jax.numpy as jnp
from jax.experimental import pallas as pl
from jax.experimental.pallas import tpu as pltpu

_LAYER_DIMS = [(784, 512), (512, 256), (256, 128), (128, 64), (64, 32), (32, 10)]
_CFG = [(4, 2), (2, 2), (2, 2), (2, 2), (2, 2), (2, 4)]
_NROUTE = [8, 4, 4, 4, 4, 8]

_BB = 2048

_RW_W = 896
_RW_OFF = np.cumsum([0] + _NROUTE)
_B_OFF = np.cumsum([0] + [d for _, d in _LAYER_DIMS])
_RB_OFF = _B_OFF[-1] + np.cumsum([0] + _NROUTE)


def _dotg(a, b, adim, bdim):
    return jax.lax.dot_general(
        a, b, (((adim,), (bdim,)), ((), ())), preferred_element_type=jnp.float32)


def _layer(cur, li, w, bcol, rw, rbcol):
    din, dout = _LAYER_DIMS[li]
    ig, og = _CFG[li]
    wi = din // ig
    wo = [dout // og] * og
    wo[-1] = dout - (og - 1) * (dout // og)
    woff = np.cumsum([0] + wo)
    first = li == 0
    cdim = 1 if first else 0

    rmask = jax.lax.broadcasted_iota(jnp.int32, w.shape, 0) == 0
    if first:
        rmask |= jax.lax.broadcasted_iota(jnp.int32, w.shape, 1) == 0
    w = jnp.where(rmask, 0.0, w)
    bcol = jnp.where(
        jax.lax.broadcasted_iota(jnp.int32, bcol.shape, 0) == 0, 0.0, bcol)

    scores = _dotg(rw, cur, 1, cdim) + rbcol
    m = jnp.max(scores, axis=0, keepdims=True)
    e = jnp.exp(scores - m)
    pw = e / jnp.sum(e, axis=0, keepdims=True)

    def cslice(i):
        return cur[:, i * wi:(i + 1) * wi] if first else cur[i * wi:(i + 1) * wi, :]

    parts = [_dotg(w[:, i * wi:(i + 1) * wi], cslice(i), 1, cdim)
             for i in range(ig)]

    psum = None
    for i in range(ig):
        s = pw[i * og:(i + 1) * og, :]
        psum = s if psum is None else psum + s

    outs = []
    for j in range(og):
        acc = _dotg(bcol[woff[j]:woff[j + 1], :], psum[j:j + 1, :], 1, 0)
        for i in range(ig):
            acc = acc + parts[i][woff[j]:woff[j + 1], :] * pw[i * og + j:i * og + j + 1, :]
        outs.append(acc)
    out = jnp.concatenate(outs, axis=0)
    if li < 5:
        out = 0.5 * out * (1.0 + jax.lax.erf(out * 0.7071067811865476))
    return out


def _body(x_ref, *refs):
    w_refs = refs[0:6]
    rw_ref = refs[6]
    b_ref = refs[7]
    o_ref = refs[8]

    cur = x_ref[...]
    for li in range(6):
        din, _ = _LAYER_DIMS[li]
        p = _NROUTE[li]
        w = w_refs[li][...]
        rw = rw_ref[_RW_OFF[li]:_RW_OFF[li] + p, 0:din]
        bcol = b_ref[_B_OFF[li]:_B_OFF[li + 1], :]
        rbcol = b_ref[_RB_OFF[li]:_RB_OFF[li + 1], :]
        cur = _layer(cur, li, w, bcol, rw, rbcol)
    o_ref[...] = jnp.swapaxes(cur, 0, 1)


def kernel(x, fc_w, fc_b, rt_w, rt_b):
    batch = x.shape[0]
    bb = _BB if batch % _BB == 0 else batch

    rw_pack = jnp.concatenate(
        [jnp.pad(r, ((0, 0), (0, _RW_W - r.shape[1]))) for r in rt_w], axis=0)
    b_pack = jnp.concatenate(
        [jnp.reshape(v, (-1, 1)) for v in list(fc_b) + list(rt_b)], axis=0)

    full = lambda arr: pl.BlockSpec(arr.shape, lambda i: (0, 0))
    in_specs = [pl.BlockSpec((bb, x.shape[1]), lambda i: (i, 0))]
    operands = [x]
    for arr in (*fc_w, rw_pack, b_pack):
        in_specs.append(full(arr))
        operands.append(arr)

    return pl.pallas_call(
        _body,
        grid=(batch // bb,),
        in_specs=in_specs,
        out_specs=pl.BlockSpec((bb, 10), lambda i: (i, 0)),
        out_shape=jax.ShapeDtypeStruct((batch, 10), jnp.float32),
        compiler_params=pltpu.CompilerParams(
            dimension_semantics=("parallel",)),
    )(*operands)
